# baseline (speedup 1.0000x reference)
import functools

import jax
import jax.numpy as jnp
from jax import lax
from jax.experimental import pallas as pl
from jax.experimental.pallas import tpu as pltpu
from jax.experimental.pallas import tpu_sc as plsc

_VOCAB = 100000
_EMBED = 64
_BATCH = 16384

_info = plsc.get_sparse_core_info()
_NC, _NS = _info.num_cores, _info.num_subcores
_NW = _NC * _NS
_RPT = _EMBED // _NW
_OCHUNK = 2048
_UNROLL = 16


@functools.partial(
    pl.kernel,
    mesh=plsc.VectorSubcoreMesh(core_axis_name="c", subcore_axis_name="s"),
    compiler_params=pltpu.CompilerParams(needs_layout_passes=False),
    out_type=(
        jax.ShapeDtypeStruct((_EMBED, _BATCH), jnp.float32),
        jax.ShapeDtypeStruct((_EMBED, _BATCH), jnp.float32),
    ),
    scratch_types=[
        pltpu.VMEM((_VOCAB,), jnp.float32),
        pltpu.VMEM((_BATCH,), jnp.int32),
        pltpu.VMEM((4 * _OCHUNK,), jnp.float32),
        pltpu.SemaphoreType.DMA,
        pltpu.SemaphoreType.DMA,
    ],
)
def _sc_rowgather(wm_hbm, wlv_hbm, idx_hbm, om_hbm, olv_hbm, row_v, idx_v, ob_v, isem, osem):
    wid = lax.axis_index("s") * _NC + lax.axis_index("c")
    icopy = pltpu.async_copy(idx_hbm, idx_v, isem)
    pending = []
    first = True
    for tbl, out, is_var in ((wm_hbm, om_hbm, False), (wlv_hbm, olv_hbm, True)):
        for r in range(_RPT):
            row = wid * _RPT + r
            pltpu.sync_copy(tbl.at[row], row_v)
            if first:
                icopy.wait()
                first = False
            for c in range(_BATCH // _OCHUNK):
                buf = len(pending) % 4
                if len(pending) >= 4:
                    pending[-4].wait()

                @plsc.parallel_loop(0, _OCHUNK, 16, unroll=_UNROLL)
                def body(i):
                    iv = idx_v[pl.ds(c * _OCHUNK + i, 16)]
                    g = plsc.load_gather(row_v, [iv])
                    if is_var:
                        g = jnp.exp(g * 2.0)
                    ob_v[pl.ds(buf * _OCHUNK + i, 16)] = g

                pending.append(
                    pltpu.async_copy(
                        ob_v.at[pl.ds(buf * _OCHUNK, _OCHUNK)],
                        out.at[row, pl.ds(c * _OCHUNK, _OCHUNK)],
                        osem,
                    )
                )
    for p in pending[-4:]:
        p.wait()


def kernel(indices, W_mean, W_log_var):
    idx = indices.astype(jnp.int32)
    gm, gv = _sc_rowgather(W_mean, W_log_var, idx)
    return gm.T, gv.T


# speedup vs baseline: 1.0754x; 1.0754x over previous
import functools

import jax
import jax.numpy as jnp
from jax import lax
from jax.experimental import pallas as pl
from jax.experimental.pallas import tpu as pltpu
from jax.experimental.pallas import tpu_sc as plsc

_VOCAB = 100000
_EMBED = 64
_BATCH = 16384

_info = plsc.get_sparse_core_info()
_NC, _NS = _info.num_cores, _info.num_subcores
_NW = _NC * _NS
_RPT = _EMBED // _NW
_OCHUNK = 4096
_UNROLL = 8


@functools.partial(
    pl.kernel,
    mesh=plsc.VectorSubcoreMesh(core_axis_name="c", subcore_axis_name="s"),
    compiler_params=pltpu.CompilerParams(needs_layout_passes=False),
    out_type=(
        jax.ShapeDtypeStruct((_EMBED, _BATCH), jnp.float32),
        jax.ShapeDtypeStruct((_EMBED, _BATCH), jnp.float32),
    ),
    scratch_types=[
        pltpu.VMEM((_VOCAB,), jnp.float32),
        pltpu.VMEM((_BATCH,), jnp.int32),
        pltpu.VMEM((2 * _OCHUNK,), jnp.float32),
        pltpu.SemaphoreType.DMA,
        pltpu.SemaphoreType.DMA,
    ],
)
def _sc_rowgather(wm_hbm, wlv_hbm, idx_hbm, om_hbm, olv_hbm, row_v, idx_v, ob_v, isem, osem):
    wid = lax.axis_index("s") * _NC + lax.axis_index("c")
    icopy = pltpu.async_copy(idx_hbm, idx_v, isem)
    pending = []
    first = True
    for tbl, out, is_var in ((wm_hbm, om_hbm, False), (wlv_hbm, olv_hbm, True)):
        for r in range(_RPT):
            row = wid * _RPT + r
            pltpu.sync_copy(tbl.at[row], row_v)
            if first:
                icopy.wait()
                first = False
            for c in range(_BATCH // _OCHUNK):
                buf = len(pending) % 2
                if len(pending) >= 2:
                    pending[-2].wait()

                @plsc.parallel_loop(0, _OCHUNK, 16, unroll=_UNROLL)
                def body(i):
                    iv = idx_v[pl.ds(c * _OCHUNK + i, 16)]
                    g = plsc.load_gather(row_v, [iv])
                    if is_var:
                        g = jnp.exp(g * 2.0)
                    ob_v[pl.ds(buf * _OCHUNK + i, 16)] = g

                pending.append(
                    pltpu.async_copy(
                        ob_v.at[pl.ds(buf * _OCHUNK, _OCHUNK)],
                        out.at[row, pl.ds(c * _OCHUNK, _OCHUNK)],
                        osem,
                    )
                )
    pending[-2].wait()
    pending[-1].wait()


def kernel(indices, W_mean, W_log_var):
    idx = indices.astype(jnp.int32)
    gm, gv = _sc_rowgather(W_mean, W_log_var, idx)
    return gm.T, gv.T


# speedup vs baseline: 1.0853x; 1.0092x over previous
import functools

import jax
import jax.numpy as jnp
from jax import lax
from jax.experimental import pallas as pl
from jax.experimental.pallas import tpu as pltpu
from jax.experimental.pallas import tpu_sc as plsc

_VOCAB = 100000
_EMBED = 64
_BATCH = 16384

_info = plsc.get_sparse_core_info()
_NC, _NS = _info.num_cores, _info.num_subcores
_NW = _NC * _NS
_RPT = _EMBED // _NW
_OCHUNK = 6144
_CHUNKS = ((0, 6144), (6144, 6144), (12288, 4096))
_UNROLL = 8


@functools.partial(
    pl.kernel,
    mesh=plsc.VectorSubcoreMesh(core_axis_name="c", subcore_axis_name="s"),
    compiler_params=pltpu.CompilerParams(needs_layout_passes=False),
    out_type=(
        jax.ShapeDtypeStruct((_EMBED, _BATCH), jnp.float32),
        jax.ShapeDtypeStruct((_EMBED, _BATCH), jnp.float32),
    ),
    scratch_types=[
        pltpu.VMEM((_VOCAB,), jnp.float32),
        pltpu.VMEM((_BATCH,), jnp.int32),
        pltpu.VMEM((2 * _OCHUNK,), jnp.float32),
        pltpu.SemaphoreType.DMA,
        pltpu.SemaphoreType.DMA,
    ],
)
def _sc_rowgather(wm_hbm, wlv_hbm, idx_hbm, om_hbm, olv_hbm, row_v, idx_v, ob_v, isem, osem):
    wid = lax.axis_index("s") * _NC + lax.axis_index("c")
    icopy = pltpu.async_copy(idx_hbm, idx_v, isem)
    pending = []
    first = True
    for tbl, out, is_var in ((wm_hbm, om_hbm, False), (wlv_hbm, olv_hbm, True)):
        for r in range(_RPT):
            row = wid * _RPT + r
            pltpu.sync_copy(tbl.at[row], row_v)
            if first:
                icopy.wait()
                first = False
            for off, size in _CHUNKS:
                buf = len(pending) % 2
                if len(pending) >= 2:
                    pending[-2].wait()

                @plsc.parallel_loop(0, size, 16, unroll=_UNROLL)
                def body(i):
                    iv = idx_v[pl.ds(off + i, 16)]
                    g = plsc.load_gather(row_v, [iv])
                    if is_var:
                        g = jnp.exp(g * 2.0)
                    ob_v[pl.ds(buf * _OCHUNK + i, 16)] = g

                pending.append(
                    pltpu.async_copy(
                        ob_v.at[pl.ds(buf * _OCHUNK, size)],
                        out.at[row, pl.ds(off, size)],
                        osem,
                    )
                )
    pending[-2].wait()
    pending[-1].wait()


def kernel(indices, W_mean, W_log_var):
    idx = indices.astype(jnp.int32)
    gm, gv = _sc_rowgather(W_mean, W_log_var, idx)
    return gm.T, gv.T
